# baseline (speedup 1.0000x reference)
import functools

import jax
import jax.numpy as jnp
from jax import lax
from jax.experimental import pallas as pl
from jax.experimental.pallas import tpu as pltpu
from jax.experimental.pallas import tpu_sc as plsc

B = 1024
L = 200
EMB = 64
FEMB = 32
OUT_D = EMB + 2 * FEMB

VOCAB = 1000000
SPLIT = 512000

N = B * L
NC = 2
NS = 16
NW = NC * NS
PER_W = N // NW
C = 128
NCH = PER_W // C
LANES = 16

TR = 2048
TSTEPS = SPLIT // TR
EDGE = VOCAB // TR


def _tr_kernel(lo_ref, hi_ref, out_ref):
    out_ref[:, 0:EMB] = lo_ref[...].T
    out_ref[:, EMB:OUT_D] = hi_ref[...].T


def _merge_transpose(wtT):
    return pl.pallas_call(
        _tr_kernel,
        grid=(TSTEPS,),
        in_specs=[
            pl.BlockSpec((EMB, TR), lambda k: (0, k)),
            pl.BlockSpec((EMB, TR),
                         lambda k: (0, jnp.where(k + TSTEPS <= EDGE,
                                                 k + TSTEPS, 0))),
        ],
        out_specs=pl.BlockSpec((TR, OUT_D), lambda k: (k, 0)),
        out_shape=jax.ShapeDtypeStruct((SPLIT, OUT_D), jnp.float32),
    )(wtT, wtT)


def _sc_gather(widx, f0idx, f1idx, wt_lin, f0t, f1t):
    mesh = plsc.VectorSubcoreMesh(core_axis_name="c", subcore_axis_name="s")

    @functools.partial(
        pl.kernel,
        out_type=jax.ShapeDtypeStruct((N, OUT_D), jnp.float32),
        mesh=mesh,
        compiler_params=pltpu.CompilerParams(use_tc_tiling_on_sc=False,
                                             needs_layout_passes=False),
        scratch_types=[
            pltpu.VMEM((PER_W,), jnp.int32),
            pltpu.VMEM((PER_W,), jnp.int32),
            pltpu.VMEM((PER_W,), jnp.int32),
            pltpu.VMEM((PER_W,), jnp.int32),
            pltpu.VMEM((PER_W,), jnp.int32),
            pltpu.VMEM((2, C, OUT_D), jnp.float32),
            pltpu.VMEM((2, C, FEMB), jnp.float32),
            pltpu.VMEM((2, C, FEMB), jnp.float32),
            pltpu.SemaphoreType.DMA,
            pltpu.SemaphoreType.DMA,
            pltpu.SemaphoreType.DMA,
            pltpu.SemaphoreType.DMA,
        ],
    )
    def k(widx_hbm, f0idx_hbm, f1idx_hbm, wt_hbm, f0t_hbm, f1t_hbm,
          out_hbm, widx_v, widx2_v, poff_v, f0idx_v, f1idx_v, gbuf,
          f0rows, f1rows, gsem0, gsem1, osem0, osem1):
        wid = lax.axis_index("s") * NC + lax.axis_index("c")
        i0 = wid * PER_W
        pltpu.sync_copy(widx_hbm.at[pl.ds(i0, PER_W)], widx_v)
        pltpu.sync_copy(f0idx_hbm.at[pl.ds(i0, PER_W)], f0idx_v)
        pltpu.sync_copy(f1idx_hbm.at[pl.ds(i0, PER_W)], f1idx_v)

        def prep(v, _):
            sl = pl.ds(v * LANES, LANES)
            x = widx_v[sl]
            hi = x >= SPLIT
            widx2_v[sl] = jnp.where(hi, x - SPLIT, x)
            poff_v[sl] = jnp.where(hi, EMB, 0)
            return 0

        lax.fori_loop(0, PER_W // LANES, prep, 0)

        def g_start(j, s, gsem):
            pltpu.async_copy(wt_hbm.at[widx2_v.at[pl.ds(j * C, C)]],
                             gbuf.at[s], gsem)
            pltpu.async_copy(f0t_hbm.at[f0idx_v.at[pl.ds(j * C, C)]],
                             f0rows.at[s], gsem)
            pltpu.async_copy(f1t_hbm.at[f1idx_v.at[pl.ds(j * C, C)]],
                             f1rows.at[s], gsem)

        def g_wait(j, s, gsem):
            pltpu.make_async_copy(wt_hbm.at[widx2_v.at[pl.ds(j * C, C)]],
                                  gbuf.at[s], gsem).wait()
            pltpu.make_async_copy(f0t_hbm.at[f0idx_v.at[pl.ds(j * C, C)]],
                                  f0rows.at[s], gsem).wait()
            pltpu.make_async_copy(f1t_hbm.at[f1idx_v.at[pl.ds(j * C, C)]],
                                  f1rows.at[s], gsem).wait()

        def assemble(j, s):
            iota = lax.iota(jnp.int32, LANES)
            g2d = gbuf.at[s]

            def group(g, _):
                pcol = poff_v[pl.ds(j * C + g * LANES, LANES)]
                rows = g * LANES + iota
                zero = pcol * 0
                for c in range(EMB):
                    x = plsc.load_gather(g2d, [rows, pcol + c])
                    plsc.store_scatter(g2d, [rows, zero + c], x)
                return 0

            lax.fori_loop(0, C // LANES, group, 0)

            def row8(r8, _):
                for rr in range(8):
                    r = r8 * 8 + rr
                    for c in range(FEMB // LANES):
                        gbuf[s, r, pl.ds(EMB + c * LANES, LANES)] = (
                            f0rows[s, r, pl.ds(c * LANES, LANES)])
                        gbuf[s, r, pl.ds(EMB + FEMB + c * LANES, LANES)] = (
                            f1rows[s, r, pl.ds(c * LANES, LANES)])
                return 0

            lax.fori_loop(0, C // 8, row8, 0)

        def o_start(j, s, osem):
            pltpu.async_copy(gbuf.at[s],
                             out_hbm.at[pl.ds(i0 + j * C, C)], osem)

        def o_wait(s, osem):
            pltpu.make_async_copy(gbuf.at[s],
                                  out_hbm.at[pl.ds(i0, C)], osem).wait()

        g_start(0, 0, gsem0)
        g_start(1, 1, gsem1)

        def body(i, _):
            a = 2 * i
            g_wait(a, 0, gsem0)
            assemble(a, 0)
            o_start(a, 0, osem0)
            g_wait(a + 1, 1, gsem1)
            assemble(a + 1, 1)
            o_start(a + 1, 1, osem1)
            o_wait(0, osem0)
            g_start(a + 2, 0, gsem0)
            o_wait(1, osem1)
            g_start(a + 3, 1, gsem1)
            return 0

        lax.fori_loop(0, (NCH - 2) // 2, body, 0)

        g_wait(NCH - 2, 0, gsem0)
        assemble(NCH - 2, 0)
        o_start(NCH - 2, 0, osem0)
        g_wait(NCH - 1, 1, gsem1)
        assemble(NCH - 1, 1)
        o_start(NCH - 1, 1, osem1)
        o_wait(0, osem0)
        o_wait(1, osem1)

    return k(widx, f0idx, f1idx, wt_lin, f0t, f1t)


@jax.jit
def _wordrep(word_inputs, feature_input_0, feature_input_1,
             word_emb_table, feat_table_0, feat_table_1):
    widx = jnp.asarray(word_inputs, jnp.int32).reshape(N)
    f0idx = jnp.asarray(feature_input_0, jnp.int32).reshape(N)
    f1idx = jnp.asarray(feature_input_1, jnp.int32).reshape(N)
    wt_lin = _merge_transpose(word_emb_table.T)
    out = _sc_gather(widx, f0idx, f1idx, wt_lin, feat_table_0, feat_table_1)
    return out.reshape(B, L, OUT_D)


def kernel(word_inputs, feature_input_0, feature_input_1,
           word_emb_table, feat_table_0, feat_table_1):
    return _wordrep(word_inputs, feature_input_0, feature_input_1,
                    word_emb_table, feat_table_0, feat_table_1)


# speedup vs baseline: 1.0000x; 1.0000x over previous
import functools

import jax
import jax.numpy as jnp
from jax import lax
from jax.experimental import pallas as pl
from jax.experimental.pallas import tpu as pltpu
from jax.experimental.pallas import tpu_sc as plsc

B = 1024
L = 200
EMB = 64
FEMB = 32
OUT_D = EMB + 2 * FEMB

VOCAB = 1000000
SPLIT = 512000

N = B * L
NC = 2
NS = 16
NW = NC * NS
PER_W = N // NW
C = 128
NCH = PER_W // C
LANES = 16

TR = 2048
TSTEPS = SPLIT // TR
EDGE = VOCAB // TR


def _tr_kernel(lo_ref, hi_ref, out_ref):
    out_ref[:, 0:EMB] = lo_ref[...].T
    out_ref[:, EMB:OUT_D] = hi_ref[...].T


def _merge_transpose(wtT):
    return pl.pallas_call(
        _tr_kernel,
        grid=(TSTEPS,),
        in_specs=[
            pl.BlockSpec((EMB, TR), lambda k: (0, k)),
            pl.BlockSpec((EMB, TR),
                         lambda k: (0, jnp.where(k + TSTEPS <= EDGE,
                                                 k + TSTEPS, 0))),
        ],
        out_specs=pl.BlockSpec((TR, OUT_D), lambda k: (k, 0)),
        out_shape=jax.ShapeDtypeStruct((SPLIT, OUT_D), jnp.float32),
    )(wtT, wtT)


def _sc_gather(widx, f0idx, f1idx, wt_lin, f0t, f1t):
    mesh = plsc.VectorSubcoreMesh(core_axis_name="c", subcore_axis_name="s")

    @functools.partial(
        pl.kernel,
        out_type=jax.ShapeDtypeStruct((N, OUT_D), jnp.float32),
        mesh=mesh,
        compiler_params=pltpu.CompilerParams(use_tc_tiling_on_sc=False,
                                             needs_layout_passes=False),
        scratch_types=[
            pltpu.VMEM((PER_W,), jnp.int32),
            pltpu.VMEM((PER_W,), jnp.int32),
            pltpu.VMEM((PER_W,), jnp.int32),
            pltpu.VMEM((PER_W,), jnp.int32),
            pltpu.VMEM((PER_W,), jnp.int32),
            pltpu.VMEM((2, C, OUT_D), jnp.float32),
            pltpu.VMEM((2, C, OUT_D), jnp.float32),
            pltpu.VMEM((2, C, FEMB), jnp.float32),
            pltpu.VMEM((2, C, FEMB), jnp.float32),
            pltpu.SemaphoreType.DMA,
            pltpu.SemaphoreType.DMA,
            pltpu.SemaphoreType.DMA,
            pltpu.SemaphoreType.DMA,
        ],
    )
    def k(widx_hbm, f0idx_hbm, f1idx_hbm, wt_hbm, f0t_hbm, f1t_hbm,
          out_hbm, widx_v, widx2_v, poff_v, f0idx_v, f1idx_v, gbuf, obuf,
          f0rows, f1rows, gsem0, gsem1, osem0, osem1):
        wid = lax.axis_index("s") * NC + lax.axis_index("c")
        i0 = wid * PER_W
        pltpu.sync_copy(widx_hbm.at[pl.ds(i0, PER_W)], widx_v)
        pltpu.sync_copy(f0idx_hbm.at[pl.ds(i0, PER_W)], f0idx_v)
        pltpu.sync_copy(f1idx_hbm.at[pl.ds(i0, PER_W)], f1idx_v)

        def prep(v, _):
            sl = pl.ds(v * LANES, LANES)
            x = widx_v[sl]
            hi = x >= SPLIT
            widx2_v[sl] = jnp.where(hi, x - SPLIT, x)
            poff_v[sl] = jnp.where(hi, EMB, 0)
            return 0

        lax.fori_loop(0, PER_W // LANES, prep, 0)

        def g_start(j, s, gsem):
            pltpu.async_copy(wt_hbm.at[widx2_v.at[pl.ds(j * C, C)]],
                             gbuf.at[s], gsem)
            pltpu.async_copy(f0t_hbm.at[f0idx_v.at[pl.ds(j * C, C)]],
                             f0rows.at[s], gsem)
            pltpu.async_copy(f1t_hbm.at[f1idx_v.at[pl.ds(j * C, C)]],
                             f1rows.at[s], gsem)

        def g_wait(j, s, gsem):
            pltpu.make_async_copy(wt_hbm.at[widx2_v.at[pl.ds(j * C, C)]],
                                  gbuf.at[s], gsem).wait()
            pltpu.make_async_copy(f0t_hbm.at[f0idx_v.at[pl.ds(j * C, C)]],
                                  f0rows.at[s], gsem).wait()
            pltpu.make_async_copy(f1t_hbm.at[f1idx_v.at[pl.ds(j * C, C)]],
                                  f1rows.at[s], gsem).wait()

        def assemble(j, s):
            iota = lax.iota(jnp.int32, LANES)
            g2d = gbuf.at[s]
            o2d = obuf.at[s]

            def group(g, _):
                pcol = poff_v[pl.ds(j * C + g * LANES, LANES)]
                rows = g * LANES + iota
                zero = pcol * 0
                for c in range(EMB):
                    x = plsc.load_gather(g2d, [rows, pcol + c])
                    plsc.store_scatter(o2d, [rows, zero + c], x)
                return 0

            lax.fori_loop(0, C // LANES, group, 0)

            def row8(r8, _):
                for rr in range(8):
                    r = r8 * 8 + rr
                    for c in range(FEMB // LANES):
                        obuf[s, r, pl.ds(EMB + c * LANES, LANES)] = (
                            f0rows[s, r, pl.ds(c * LANES, LANES)])
                        obuf[s, r, pl.ds(EMB + FEMB + c * LANES, LANES)] = (
                            f1rows[s, r, pl.ds(c * LANES, LANES)])
                return 0

            lax.fori_loop(0, C // 8, row8, 0)

        def o_start(j, s, osem):
            pltpu.async_copy(obuf.at[s],
                             out_hbm.at[pl.ds(i0 + j * C, C)], osem)

        def o_wait(s, osem):
            pltpu.make_async_copy(obuf.at[s],
                                  out_hbm.at[pl.ds(i0, C)], osem).wait()

        g_start(0, 0, gsem0)
        g_start(1, 1, gsem1)

        def body(i, _):
            a = 2 * i
            g_wait(a, 0, gsem0)
            assemble(a, 0)
            o_start(a, 0, osem0)
            g_wait(a + 1, 1, gsem1)
            assemble(a + 1, 1)
            o_start(a + 1, 1, osem1)
            o_wait(0, osem0)
            g_start(a + 2, 0, gsem0)
            o_wait(1, osem1)
            g_start(a + 3, 1, gsem1)
            return 0

        lax.fori_loop(0, (NCH - 2) // 2, body, 0)

        g_wait(NCH - 2, 0, gsem0)
        assemble(NCH - 2, 0)
        o_start(NCH - 2, 0, osem0)
        g_wait(NCH - 1, 1, gsem1)
        assemble(NCH - 1, 1)
        o_start(NCH - 1, 1, osem1)
        o_wait(0, osem0)
        o_wait(1, osem1)

    return k(widx, f0idx, f1idx, wt_lin, f0t, f1t)


@jax.jit
def _wordrep(word_inputs, feature_input_0, feature_input_1,
             word_emb_table, feat_table_0, feat_table_1):
    widx = jnp.asarray(word_inputs, jnp.int32).reshape(N)
    f0idx = jnp.asarray(feature_input_0, jnp.int32).reshape(N)
    f1idx = jnp.asarray(feature_input_1, jnp.int32).reshape(N)
    wt_lin = _merge_transpose(word_emb_table.T)
    out = _sc_gather(widx, f0idx, f1idx, wt_lin, feat_table_0, feat_table_1)
    return out.reshape(B, L, OUT_D)


def kernel(word_inputs, feature_input_0, feature_input_1,
           word_emb_table, feat_table_0, feat_table_1):
    return _wordrep(word_inputs, feature_input_0, feature_input_1,
                    word_emb_table, feat_table_0, feat_table_1)


# speedup vs baseline: 1.3053x; 1.3053x over previous
import functools

import jax
import jax.numpy as jnp
from jax import lax
from jax.experimental import pallas as pl
from jax.experimental.pallas import tpu as pltpu
from jax.experimental.pallas import tpu_sc as plsc

B = 1024
L = 200
EMB = 64
FEMB = 32
OUT_D = EMB + 2 * FEMB

VOCAB = 1000000
SPLIT = 512000

N = B * L
NC = 2
NS = 16
NW = NC * NS
PER_W = N // NW
C = 128
NCH = PER_W // C
LANES = 16

TR = 2048
TSTEPS = SPLIT // TR
EDGE = VOCAB // TR


def _tr_kernel(lo_ref, hi_ref, out_ref):
    out_ref[:, 0:EMB] = lo_ref[...].T
    out_ref[:, EMB:OUT_D] = hi_ref[...].T


def _merge_transpose(wtT):
    return pl.pallas_call(
        _tr_kernel,
        grid=(TSTEPS,),
        in_specs=[
            pl.BlockSpec((EMB, TR), lambda k: (0, k)),
            pl.BlockSpec((EMB, TR),
                         lambda k: (0, jnp.where(k + TSTEPS <= EDGE,
                                                 k + TSTEPS, 0))),
        ],
        out_specs=pl.BlockSpec((TR, OUT_D), lambda k: (k, 0)),
        out_shape=jax.ShapeDtypeStruct((SPLIT, OUT_D), jnp.float32),
    )(wtT, wtT)


def _sc_gather(widx, f0idx, f1idx, wt_lin, f0t, f1t):
    mesh = plsc.VectorSubcoreMesh(core_axis_name="c", subcore_axis_name="s")

    @functools.partial(
        pl.kernel,
        out_type=jax.ShapeDtypeStruct((N, OUT_D), jnp.float32),
        mesh=mesh,
        compiler_params=pltpu.CompilerParams(use_tc_tiling_on_sc=False,
                                             needs_layout_passes=False),
        scratch_types=[
            pltpu.VMEM((PER_W,), jnp.int32),
            pltpu.VMEM((PER_W,), jnp.int32),
            pltpu.VMEM((PER_W,), jnp.int32),
            pltpu.VMEM((PER_W,), jnp.int32),
            pltpu.VMEM((PER_W,), jnp.int32),
            pltpu.VMEM((2, C, OUT_D), jnp.float32),
            pltpu.VMEM((2, C, OUT_D), jnp.float32),
            pltpu.VMEM((2, C, FEMB), jnp.float32),
            pltpu.VMEM((2, C, FEMB), jnp.float32),
            pltpu.SemaphoreType.DMA,
            pltpu.SemaphoreType.DMA,
            pltpu.SemaphoreType.DMA,
            pltpu.SemaphoreType.DMA,
        ],
    )
    def k(widx_hbm, f0idx_hbm, f1idx_hbm, wt_hbm, f0t_hbm, f1t_hbm,
          out_hbm, widx_v, widx2_v, poff_v, f0idx_v, f1idx_v, gbuf, obuf,
          f0rows, f1rows, gsem0, gsem1, osem0, osem1):
        wid = lax.axis_index("s") * NC + lax.axis_index("c")
        i0 = wid * PER_W
        pltpu.sync_copy(widx_hbm.at[pl.ds(i0, PER_W)], widx_v)
        pltpu.sync_copy(f0idx_hbm.at[pl.ds(i0, PER_W)], f0idx_v)
        pltpu.sync_copy(f1idx_hbm.at[pl.ds(i0, PER_W)], f1idx_v)

        def prep(v, _):
            sl = pl.ds(v * LANES, LANES)
            x = widx_v[sl]
            hi = x >= SPLIT
            widx2_v[sl] = jnp.where(hi, x - SPLIT, x)
            poff_v[sl] = jnp.where(hi, EMB, 0)
            return 0

        lax.fori_loop(0, PER_W // LANES, prep, 0)

        def g_start(j, s, gsem):
            pltpu.async_copy(wt_hbm.at[widx2_v.at[pl.ds(j * C, C)]],
                             gbuf.at[s], gsem)
            pltpu.async_copy(f0t_hbm.at[f0idx_v.at[pl.ds(j * C, C)]],
                             f0rows.at[s], gsem)
            pltpu.async_copy(f1t_hbm.at[f1idx_v.at[pl.ds(j * C, C)]],
                             f1rows.at[s], gsem)

        def g_wait(j, s, gsem):
            pltpu.make_async_copy(wt_hbm.at[widx2_v.at[pl.ds(j * C, C)]],
                                  gbuf.at[s], gsem).wait()
            pltpu.make_async_copy(f0t_hbm.at[f0idx_v.at[pl.ds(j * C, C)]],
                                  f0rows.at[s], gsem).wait()
            pltpu.make_async_copy(f1t_hbm.at[f1idx_v.at[pl.ds(j * C, C)]],
                                  f1rows.at[s], gsem).wait()

        def assemble(j, s):
            iota = lax.iota(jnp.int32, LANES)
            g2d = gbuf.at[s]
            o2d = obuf.at[s]

            def group(g, _):
                pcol = poff_v[pl.ds(j * C + g * LANES, LANES)]
                rows = g * LANES + iota
                zero = pcol * 0

                @plsc.parallel_loop(0, EMB, unroll=8)
                def _(c):
                    x = plsc.load_gather(g2d, [rows, pcol + c])
                    plsc.store_scatter(o2d, [rows, zero + c], x)

                return 0

            lax.fori_loop(0, C // LANES, group, 0)

            @plsc.parallel_loop(0, C, unroll=8)
            def _(r):
                for c in range(FEMB // LANES):
                    obuf[s, r, pl.ds(EMB + c * LANES, LANES)] = (
                        f0rows[s, r, pl.ds(c * LANES, LANES)])
                    obuf[s, r, pl.ds(EMB + FEMB + c * LANES, LANES)] = (
                        f1rows[s, r, pl.ds(c * LANES, LANES)])

        def o_start(j, s, osem):
            pltpu.async_copy(obuf.at[s],
                             out_hbm.at[pl.ds(i0 + j * C, C)], osem)

        def o_wait(s, osem):
            pltpu.make_async_copy(obuf.at[s],
                                  out_hbm.at[pl.ds(i0, C)], osem).wait()

        g_start(0, 0, gsem0)
        g_start(1, 1, gsem1)

        def body(i, _):
            a = 2 * i
            g_wait(a, 0, gsem0)
            assemble(a, 0)
            o_start(a, 0, osem0)
            g_wait(a + 1, 1, gsem1)
            assemble(a + 1, 1)
            o_start(a + 1, 1, osem1)
            o_wait(0, osem0)
            g_start(a + 2, 0, gsem0)
            o_wait(1, osem1)
            g_start(a + 3, 1, gsem1)
            return 0

        lax.fori_loop(0, (NCH - 2) // 2, body, 0)

        g_wait(NCH - 2, 0, gsem0)
        assemble(NCH - 2, 0)
        o_start(NCH - 2, 0, osem0)
        g_wait(NCH - 1, 1, gsem1)
        assemble(NCH - 1, 1)
        o_start(NCH - 1, 1, osem1)
        o_wait(0, osem0)
        o_wait(1, osem1)

    return k(widx, f0idx, f1idx, wt_lin, f0t, f1t)


@jax.jit
def _wordrep(word_inputs, feature_input_0, feature_input_1,
             word_emb_table, feat_table_0, feat_table_1):
    widx = jnp.asarray(word_inputs, jnp.int32).reshape(N)
    f0idx = jnp.asarray(feature_input_0, jnp.int32).reshape(N)
    f1idx = jnp.asarray(feature_input_1, jnp.int32).reshape(N)
    wt_lin = _merge_transpose(word_emb_table.T)
    out = _sc_gather(widx, f0idx, f1idx, wt_lin, feat_table_0, feat_table_1)
    return out.reshape(B, L, OUT_D)


def kernel(word_inputs, feature_input_0, feature_input_1,
           word_emb_table, feat_table_0, feat_table_1):
    return _wordrep(word_inputs, feature_input_0, feature_input_1,
                    word_emb_table, feat_table_0, feat_table_1)
